# baseline (speedup 1.0000x reference)
import functools

import jax
import jax.numpy as jnp
from jax import lax
from jax.experimental import pallas as pl
from jax.experimental.pallas import tpu as pltpu
from jax.experimental.pallas import tpu_sc as plsc

N = 10000
E = 320000
F = 128
H = 8
C = 16
NCLS = 40

NW = 32
BLK = 100
TPB = 100
IDXC = 10
NBUF = 2
EP = NW * TPB * BLK
ROWS2D = EP // BLK
ACC_R = 10112
DUMMY = N
RPT = ACC_R // 16
TW = F
AW = F + C

_BN = 2000
_GRID = N // _BN


def _tc_proj_body(x_ref, w_ref, ss_ref, sd_ref, xw_ref, a_ref, b_ref, m_ref):
    xw = jnp.dot(x_ref[...], w_ref[...], preferred_element_type=jnp.float32)
    a = jnp.dot(xw, ss_ref[...], preferred_element_type=jnp.float32)
    b = jnp.dot(xw, sd_ref[...], preferred_element_type=jnp.float32)
    xw_ref[...] = xw
    a_ref[...] = a
    b_ref[...] = b
    cur = jnp.concatenate(
        [jnp.max(a).reshape(1, 1), jnp.max(b).reshape(1, 1)], axis=1)
    i = pl.program_id(0)

    @pl.when(i == 0)
    def _():
        m_ref[...] = cur

    @pl.when(i > 0)
    def _():
        m_ref[...] = jnp.maximum(m_ref[...], cur)


def _tc_proj(x, w, ss, sd):
    fw = x.shape[1]
    return pl.pallas_call(
        _tc_proj_body,
        grid=(_GRID,),
        in_specs=[
            pl.BlockSpec((_BN, fw), lambda i: (i, 0)),
            pl.BlockSpec((fw, F), lambda i: (0, 0)),
            pl.BlockSpec((F, C), lambda i: (0, 0)),
            pl.BlockSpec((F, C), lambda i: (0, 0)),
        ],
        out_specs=[
            pl.BlockSpec((_BN, F), lambda i: (i, 0)),
            pl.BlockSpec((_BN, C), lambda i: (i, 0)),
            pl.BlockSpec((_BN, C), lambda i: (i, 0)),
            pl.BlockSpec((1, 2), lambda i: (0, 0)),
        ],
        out_shape=[
            jax.ShapeDtypeStruct((N, F), jnp.float32),
            jax.ShapeDtypeStruct((N, C), jnp.float32),
            jax.ShapeDtypeStruct((N, C), jnp.float32),
            jax.ShapeDtypeStruct((1, 2), jnp.float32),
        ],
    )(x, w, ss, sd)


def _tc_comb_body(apply_elu, has_att, p0_ref, p1_ref, e8_ref, bias_ref, w_ref,
                  ss_ref, sd_ref, xw_ref, a_ref, b_ref, m_ref):
    acc = p0_ref[...] + p1_ref[...]
    num = acc[:, :F]
    den = acc[:, F:F + H]
    dene = jnp.dot(den, e8_ref[...], preferred_element_type=jnp.float32)
    h = num / (dene + 1e-16) + bias_ref[...]
    if apply_elu:
        h = jnp.where(h > 0.0, h, jnp.exp(h) - 1.0)
    xw = jnp.dot(h, w_ref[...], preferred_element_type=jnp.float32)
    xw_ref[...] = xw
    if has_att:
        a = jnp.dot(xw, ss_ref[...], preferred_element_type=jnp.float32)
        b = jnp.dot(xw, sd_ref[...], preferred_element_type=jnp.float32)
        a_ref[...] = a
        b_ref[...] = b
        cur = jnp.concatenate(
            [jnp.max(a).reshape(1, 1), jnp.max(b).reshape(1, 1)], axis=1)
        i = pl.program_id(0)

        @pl.when(i == 0)
        def _():
            m_ref[...] = cur

        @pl.when(i > 0)
        def _():
            m_ref[...] = jnp.maximum(m_ref[...], cur)


def _tc_combine(p0, p1, e8, bias, w, ss, sd, apply_elu):
    fw = w.shape[1]
    body = functools.partial(_tc_comb_body, apply_elu, True)
    return pl.pallas_call(
        body,
        grid=(_GRID,),
        in_specs=[
            pl.BlockSpec((_BN, AW), lambda i: (i, 0)),
            pl.BlockSpec((_BN, AW), lambda i: (i, 0)),
            pl.BlockSpec((H, F), lambda i: (0, 0)),
            pl.BlockSpec((1, F), lambda i: (0, 0)),
            pl.BlockSpec((F, fw), lambda i: (0, 0)),
            pl.BlockSpec((fw, C), lambda i: (0, 0)),
            pl.BlockSpec((fw, C), lambda i: (0, 0)),
        ],
        out_specs=[
            pl.BlockSpec((_BN, fw), lambda i: (i, 0)),
            pl.BlockSpec((_BN, C), lambda i: (i, 0)),
            pl.BlockSpec((_BN, C), lambda i: (i, 0)),
            pl.BlockSpec((1, 2), lambda i: (0, 0)),
        ],
        out_shape=[
            jax.ShapeDtypeStruct((N, fw), jnp.float32),
            jax.ShapeDtypeStruct((N, C), jnp.float32),
            jax.ShapeDtypeStruct((N, C), jnp.float32),
            jax.ShapeDtypeStruct((1, 2), jnp.float32),
        ],
    )(p0, p1, e8, bias, w, ss, sd)


def _tc_final_body(p0_ref, p1_ref, e8_ref, bias_ref, w_ref, bc_ref, out_ref):
    acc = p0_ref[...] + p1_ref[...]
    num = acc[:, :F]
    den = acc[:, F:F + H]
    dene = jnp.dot(den, e8_ref[...], preferred_element_type=jnp.float32)
    h = num / (dene + 1e-16) + bias_ref[...]
    out_ref[...] = (
        jnp.dot(h, w_ref[...], preferred_element_type=jnp.float32)
        + bc_ref[...])


def _tc_final(p0, p1, e8, bias, wc_pad, bc_pad):
    return pl.pallas_call(
        _tc_final_body,
        grid=(_GRID,),
        in_specs=[
            pl.BlockSpec((_BN, AW), lambda i: (i, 0)),
            pl.BlockSpec((_BN, AW), lambda i: (i, 0)),
            pl.BlockSpec((H, F), lambda i: (0, 0)),
            pl.BlockSpec((1, F), lambda i: (0, 0)),
            pl.BlockSpec((F, F), lambda i: (0, 0)),
            pl.BlockSpec((1, F), lambda i: (0, 0)),
        ],
        out_specs=pl.BlockSpec((_BN, F), lambda i: (i, 0)),
        out_shape=jax.ShapeDtypeStruct((N, F), jnp.float32),
    )(p0, p1, e8, bias, wc_pad, bc_pad)


def _sc_edge_body(t2_hbm, at_hbm, bt_hbm, src_hbm, dst_hbm, g_hbm, out_hbm,
                  acc_sh, src_v, dst_v, m0_v, g_v,
                  rows0_v, rows1_v, av0_v, av1_v, bv0_v, bv1_v,
                  sem0, sem1):
    c = lax.axis_index("c")
    s = lax.axis_index("s")
    wid = s * 2 + c
    rows_b = [rows0_v, rows1_v]
    av_b = [av0_v, av1_v]
    bv_b = [bv0_v, bv1_v]
    sems = [sem0, sem1]

    zero16 = jnp.zeros((16,), jnp.float32)

    def zrow(i, carry):
        for g in range(AW // 16):
            m0_v[i, pl.ds(g * 16, 16)] = zero16
        return carry

    lax.fori_loop(0, BLK, zrow, 0)
    base = s * RPT

    def zacc(j, carry):
        pltpu.sync_copy(m0_v, acc_sh.at[pl.ds(base + j * BLK, BLK)])
        return carry

    lax.fori_loop(0, RPT // BLK, zacc, 0)
    pltpu.sync_copy(m0_v.at[pl.ds(0, RPT % BLK)],
                    acc_sh.at[pl.ds(base + (RPT // BLK) * BLK, RPT % BLK)])
    plsc.subcore_barrier()

    pltpu.sync_copy(g_hbm, g_v)
    gv = g_v[...]
    maskv = jnp.where(lax.iota(jnp.int32, 16) < H, 1.0, 0.0)

    def _compute(rows_v, av_v, bv_v, m_v):
        def edge(b, inner):
            alpha = av_v[b, :] + bv_v[b, :]
            alpha = jnp.where(alpha > 0.0, alpha, alpha * 0.2)
            w = jnp.exp(alpha - gv)
            for g in range(H // 2):
                v = rows_v[b, pl.ds(32 * g, 32)]
                vi = plsc.bitcast(v, jnp.int32)
                a16 = plsc.bitcast(lax.shift_left(vi, 16), jnp.float32)
                b16 = plsc.bitcast(vi & jnp.int32(-65536), jnp.float32)
                m_v[b, pl.ds(2 * g * 16, 16)] = a16 * w[2 * g]
                m_v[b, pl.ds((2 * g + 1) * 16, 16)] = b16 * w[2 * g + 1]
            m_v[b, pl.ds(F, 16)] = w * maskv
            return inner

        lax.fori_loop(0, BLK, edge, 0, unroll=10)

    def chunk(kk, carry):
        row0 = wid * TPB + kk * IDXC
        pltpu.sync_copy(src_hbm.at[pl.ds(row0, IDXC)], src_v)
        pltpu.sync_copy(dst_hbm.at[pl.ds(row0, IDXC)], dst_v)
        for q in range(NBUF):
            pltpu.async_copy(t2_hbm.at[src_v.at[q]], rows_b[q], sems[q])
            pltpu.async_copy(at_hbm.at[src_v.at[q]], av_b[q], sems[q])
            pltpu.async_copy(bt_hbm.at[dst_v.at[q]], bv_b[q], sems[q])

        def slot(rows_v, av_v, bv_v, sem, k_self, k_next):
            pltpu.make_async_copy(
                t2_hbm.at[src_v.at[k_self]], rows_v, sem).wait()
            pltpu.make_async_copy(
                at_hbm.at[src_v.at[k_self]], av_v, sem).wait()
            pltpu.make_async_copy(
                bt_hbm.at[dst_v.at[k_self]], bv_v, sem).wait()

            _compute(rows_v, av_v, bv_v, m0_v)

            @pl.when(k_next < IDXC)
            def _():
                pltpu.async_copy(t2_hbm.at[src_v.at[k_next]], rows_v, sem)
                pltpu.async_copy(at_hbm.at[src_v.at[k_next]], av_v, sem)
                pltpu.async_copy(bt_hbm.at[dst_v.at[k_next]], bv_v, sem)

            pltpu.sync_copy(m0_v, acc_sh.at[dst_v.at[k_self]], add=True)

        def ring(i, c2):
            for q in range(NBUF):
                slot(rows_b[q], av_b[q], bv_b[q], sems[q],
                     NBUF * i + q, NBUF * i + q + NBUF)
            return c2

        lax.fori_loop(0, IDXC // NBUF, ring, 0)
        return carry

    lax.fori_loop(0, TPB // IDXC, chunk, 0)
    plsc.subcore_barrier()

    ob = c * ACC_R + s * RPT

    def wb(j, carry):
        pltpu.sync_copy(acc_sh.at[pl.ds(base + j * BLK, BLK)],
                        out_hbm.at[pl.ds(ob + j * BLK, BLK)])
        return carry

    lax.fori_loop(0, RPT // BLK, wb, 0)
    pltpu.sync_copy(acc_sh.at[pl.ds(base + (RPT // BLK) * BLK, RPT % BLK)],
                    out_hbm.at[pl.ds(ob + (RPT // BLK) * BLK, RPT % BLK)])


_sc_edge = pl.kernel(
    _sc_edge_body,
    out_type=jax.ShapeDtypeStruct((2 * ACC_R, AW), jnp.float32),
    mesh=plsc.VectorSubcoreMesh(
        core_axis_name="c", subcore_axis_name="s",
        num_cores=2, num_subcores=16),
    compiler_params=pltpu.CompilerParams(
        use_tc_tiling_on_sc=False, needs_layout_passes=False),
    scratch_types=[
        pltpu.VMEM_SHARED((ACC_R, AW), jnp.float32),
        pltpu.VMEM((IDXC, BLK), jnp.int32),
        pltpu.VMEM((IDXC, BLK), jnp.int32),
        pltpu.VMEM((BLK, AW), jnp.float32),
        pltpu.VMEM((16,), jnp.float32),
        pltpu.VMEM((BLK, TW), jnp.bfloat16),
        pltpu.VMEM((BLK, TW), jnp.bfloat16),
        pltpu.VMEM((BLK, C), jnp.float32),
        pltpu.VMEM((BLK, C), jnp.float32),
        pltpu.VMEM((BLK, C), jnp.float32),
        pltpu.VMEM((BLK, C), jnp.float32),
        pltpu.SemaphoreType.DMA,
        pltpu.SemaphoreType.DMA,
    ],
)


def _att_mat(att):
    af = att.reshape(F)
    idx = jnp.arange(F) // C
    onehot = (idx[:, None] == jnp.arange(C)[None, :]).astype(jnp.float32)
    return af[:, None] * onehot


def _gat_layer(xw, a, b, m, srcp, dstp):
    g = jnp.maximum(m[0, 0] + m[0, 1], 0.0)
    gvec = jnp.full((16,), g, jnp.float32)
    t2b = (xw.reshape(N, H // 2, 2, C).transpose(0, 1, 3, 2)
           .reshape(N, F).astype(jnp.bfloat16))
    acc = _sc_edge(t2b, a, b, srcp, dstp, gvec)
    return acc[:N], acc[ACC_R:ACC_R + N]


def kernel(x, edge_index, W0, att_src0, att_dst0, b0,
           W1, att_src1, att_dst1, b1, Wc, bc):
    f32 = jnp.float32
    ss0 = _att_mat(att_src0)
    sd0 = _att_mat(att_dst0)
    ss1 = _att_mat(att_src1)
    sd1 = _att_mat(att_dst1)
    e8 = (jnp.arange(F)[None, :] // C == jnp.arange(H)[:, None]).astype(f32)

    src = edge_index[0]
    dst = edge_index[1]
    pad = EP - E
    srcp = jnp.concatenate(
        [src, jnp.zeros((pad,), jnp.int32)]).reshape(ROWS2D, BLK)
    dstp = jnp.concatenate(
        [dst, jnp.full((pad,), DUMMY, jnp.int32)]).reshape(ROWS2D, BLK)

    xw0, a0, bt0, m0 = _tc_proj(x, W0, ss0, sd0)
    p00, p01 = _gat_layer(xw0, a0, bt0, m0, srcp, dstp)

    xw1, a1, bt1, m1 = _tc_combine(
        p00, p01, e8, b0.reshape(1, F), W1, ss1, sd1, apply_elu=True)
    p10, p11 = _gat_layer(xw1, a1, bt1, m1, srcp, dstp)

    wc_pad = jnp.zeros((F, F), f32).at[:, :NCLS].set(Wc)
    bc_pad = jnp.zeros((1, F), f32).at[0, :NCLS].set(bc)
    logits = _tc_final(p10, p11, e8, b1.reshape(1, F), wc_pad, bc_pad)
    return logits[:, :NCLS]


# speedup vs baseline: 1.0188x; 1.0188x over previous
import functools

import jax
import jax.numpy as jnp
from jax import lax
from jax.experimental import pallas as pl
from jax.experimental.pallas import tpu as pltpu
from jax.experimental.pallas import tpu_sc as plsc

N = 10000
E = 320000
F = 128
H = 8
C = 16
NCLS = 40

NW = 32
BLK = 100
TPB = 100
IDXC = 10
NBUF = 2
EP = NW * TPB * BLK
ROWS2D = EP // BLK
ACC_R = 10112
DUMMY = N
RPT = ACC_R // 16
TW = F
AW = F + C

_BN = 2000
_GRID = N // _BN


def _tc_proj_body(x_ref, w_ref, ss_ref, sd_ref, xw_ref, a_ref, b_ref, m_ref):
    xw = jnp.dot(x_ref[...], w_ref[...], preferred_element_type=jnp.float32)
    a = jnp.dot(xw, ss_ref[...], preferred_element_type=jnp.float32)
    b = jnp.dot(xw, sd_ref[...], preferred_element_type=jnp.float32)
    xw_ref[...] = xw
    a_ref[...] = a
    b_ref[...] = b
    cur = jnp.concatenate(
        [jnp.max(a).reshape(1, 1), jnp.max(b).reshape(1, 1)], axis=1)
    i = pl.program_id(0)

    @pl.when(i == 0)
    def _():
        m_ref[...] = cur

    @pl.when(i > 0)
    def _():
        m_ref[...] = jnp.maximum(m_ref[...], cur)


def _tc_proj(x, w, ss, sd):
    fw = x.shape[1]
    return pl.pallas_call(
        _tc_proj_body,
        grid=(_GRID,),
        in_specs=[
            pl.BlockSpec((_BN, fw), lambda i: (i, 0)),
            pl.BlockSpec((fw, F), lambda i: (0, 0)),
            pl.BlockSpec((F, C), lambda i: (0, 0)),
            pl.BlockSpec((F, C), lambda i: (0, 0)),
        ],
        out_specs=[
            pl.BlockSpec((_BN, F), lambda i: (i, 0)),
            pl.BlockSpec((_BN, C), lambda i: (i, 0)),
            pl.BlockSpec((_BN, C), lambda i: (i, 0)),
            pl.BlockSpec((1, 2), lambda i: (0, 0)),
        ],
        out_shape=[
            jax.ShapeDtypeStruct((N, F), jnp.float32),
            jax.ShapeDtypeStruct((N, C), jnp.float32),
            jax.ShapeDtypeStruct((N, C), jnp.float32),
            jax.ShapeDtypeStruct((1, 2), jnp.float32),
        ],
    )(x, w, ss, sd)


def _tc_comb_body(apply_elu, has_att, p0_ref, p1_ref, e8_ref, bias_ref, w_ref,
                  ss_ref, sd_ref, xw_ref, a_ref, b_ref, m_ref):
    acc = p0_ref[...] + p1_ref[...]
    num = acc[:, :F]
    den = acc[:, F:F + H]
    dene = jnp.dot(den, e8_ref[...], preferred_element_type=jnp.float32)
    h = num / (dene + 1e-16) + bias_ref[...]
    if apply_elu:
        h = jnp.where(h > 0.0, h, jnp.exp(h) - 1.0)
    xw = jnp.dot(h, w_ref[...], preferred_element_type=jnp.float32)
    xw_ref[...] = xw
    if has_att:
        a = jnp.dot(xw, ss_ref[...], preferred_element_type=jnp.float32)
        b = jnp.dot(xw, sd_ref[...], preferred_element_type=jnp.float32)
        a_ref[...] = a
        b_ref[...] = b
        cur = jnp.concatenate(
            [jnp.max(a).reshape(1, 1), jnp.max(b).reshape(1, 1)], axis=1)
        i = pl.program_id(0)

        @pl.when(i == 0)
        def _():
            m_ref[...] = cur

        @pl.when(i > 0)
        def _():
            m_ref[...] = jnp.maximum(m_ref[...], cur)


def _tc_combine(p0, p1, e8, bias, w, ss, sd, apply_elu):
    fw = w.shape[1]
    body = functools.partial(_tc_comb_body, apply_elu, True)
    return pl.pallas_call(
        body,
        grid=(_GRID,),
        in_specs=[
            pl.BlockSpec((_BN, AW), lambda i: (i, 0)),
            pl.BlockSpec((_BN, AW), lambda i: (i, 0)),
            pl.BlockSpec((H, F), lambda i: (0, 0)),
            pl.BlockSpec((1, F), lambda i: (0, 0)),
            pl.BlockSpec((F, fw), lambda i: (0, 0)),
            pl.BlockSpec((fw, C), lambda i: (0, 0)),
            pl.BlockSpec((fw, C), lambda i: (0, 0)),
        ],
        out_specs=[
            pl.BlockSpec((_BN, fw), lambda i: (i, 0)),
            pl.BlockSpec((_BN, C), lambda i: (i, 0)),
            pl.BlockSpec((_BN, C), lambda i: (i, 0)),
            pl.BlockSpec((1, 2), lambda i: (0, 0)),
        ],
        out_shape=[
            jax.ShapeDtypeStruct((N, fw), jnp.float32),
            jax.ShapeDtypeStruct((N, C), jnp.float32),
            jax.ShapeDtypeStruct((N, C), jnp.float32),
            jax.ShapeDtypeStruct((1, 2), jnp.float32),
        ],
    )(p0, p1, e8, bias, w, ss, sd)


def _tc_final_body(p0_ref, p1_ref, e8_ref, bias_ref, w_ref, bc_ref, out_ref):
    acc = p0_ref[...] + p1_ref[...]
    num = acc[:, :F]
    den = acc[:, F:F + H]
    dene = jnp.dot(den, e8_ref[...], preferred_element_type=jnp.float32)
    h = num / (dene + 1e-16) + bias_ref[...]
    out_ref[...] = (
        jnp.dot(h, w_ref[...], preferred_element_type=jnp.float32)
        + bc_ref[...])


def _tc_final(p0, p1, e8, bias, wc_pad, bc_pad):
    return pl.pallas_call(
        _tc_final_body,
        grid=(_GRID,),
        in_specs=[
            pl.BlockSpec((_BN, AW), lambda i: (i, 0)),
            pl.BlockSpec((_BN, AW), lambda i: (i, 0)),
            pl.BlockSpec((H, F), lambda i: (0, 0)),
            pl.BlockSpec((1, F), lambda i: (0, 0)),
            pl.BlockSpec((F, F), lambda i: (0, 0)),
            pl.BlockSpec((1, F), lambda i: (0, 0)),
        ],
        out_specs=pl.BlockSpec((_BN, F), lambda i: (i, 0)),
        out_shape=jax.ShapeDtypeStruct((N, F), jnp.float32),
    )(p0, p1, e8, bias, wc_pad, bc_pad)


def _sc_edge_body(t2_hbm, at_hbm, bt_hbm, src_hbm, dst_hbm, g_hbm, out_hbm,
                  acc_sh, src_v, dst_v, m0_v, g_v,
                  rows0_v, rows1_v, av0_v, av1_v, bv0_v, bv1_v,
                  sem0, sem1):
    c = lax.axis_index("c")
    s = lax.axis_index("s")
    wid = s * 2 + c
    rows_b = [rows0_v, rows1_v]
    av_b = [av0_v, av1_v]
    bv_b = [bv0_v, bv1_v]
    sems = [sem0, sem1]

    zero16 = jnp.zeros((16,), jnp.float32)

    def zrow(i, carry):
        for g in range(AW // 16):
            m0_v[i, pl.ds(g * 16, 16)] = zero16
        return carry

    lax.fori_loop(0, BLK, zrow, 0)
    base = s * RPT

    def zacc(j, carry):
        pltpu.sync_copy(m0_v, acc_sh.at[pl.ds(base + j * BLK, BLK)])
        return carry

    lax.fori_loop(0, RPT // BLK, zacc, 0)
    pltpu.sync_copy(m0_v.at[pl.ds(0, RPT % BLK)],
                    acc_sh.at[pl.ds(base + (RPT // BLK) * BLK, RPT % BLK)])
    plsc.subcore_barrier()

    pltpu.sync_copy(g_hbm, g_v)
    gv = g_v[...]
    maskv = jnp.where(lax.iota(jnp.int32, 16) < H, 1.0, 0.0)

    def _compute(rows_v, av_v, bv_v, m_v):
        def edge(b, inner):
            alpha = av_v[b, :] + bv_v[b, :]
            alpha = jnp.where(alpha > 0.0, alpha, alpha * 0.2)
            w = jnp.exp(alpha - gv)
            for g in range(H // 2):
                v = rows_v[b, pl.ds(32 * g, 32)]
                vi = plsc.bitcast(v, jnp.int32)
                a16 = plsc.bitcast(lax.shift_left(vi, 16), jnp.float32)
                b16 = plsc.bitcast(vi & jnp.int32(-65536), jnp.float32)
                m_v[b, pl.ds(2 * g * 16, 16)] = a16 * w[2 * g]
                m_v[b, pl.ds((2 * g + 1) * 16, 16)] = b16 * w[2 * g + 1]
            m_v[b, pl.ds(F, 16)] = w * maskv
            return inner

        lax.fori_loop(0, BLK, edge, 0, unroll=8)

    def chunk(kk, carry):
        row0 = wid * TPB + kk * IDXC
        pltpu.sync_copy(src_hbm.at[pl.ds(row0, IDXC)], src_v)
        pltpu.sync_copy(dst_hbm.at[pl.ds(row0, IDXC)], dst_v)
        for q in range(NBUF):
            pltpu.async_copy(t2_hbm.at[src_v.at[q]], rows_b[q], sems[q])
            pltpu.async_copy(at_hbm.at[src_v.at[q]], av_b[q], sems[q])
            pltpu.async_copy(bt_hbm.at[dst_v.at[q]], bv_b[q], sems[q])

        def slot(rows_v, av_v, bv_v, sem, k_self, k_next):
            pltpu.make_async_copy(
                t2_hbm.at[src_v.at[k_self]], rows_v, sem).wait()
            pltpu.make_async_copy(
                at_hbm.at[src_v.at[k_self]], av_v, sem).wait()
            pltpu.make_async_copy(
                bt_hbm.at[dst_v.at[k_self]], bv_v, sem).wait()

            _compute(rows_v, av_v, bv_v, m0_v)

            @pl.when(k_next < IDXC)
            def _():
                pltpu.async_copy(t2_hbm.at[src_v.at[k_next]], rows_v, sem)
                pltpu.async_copy(at_hbm.at[src_v.at[k_next]], av_v, sem)
                pltpu.async_copy(bt_hbm.at[dst_v.at[k_next]], bv_v, sem)

            pltpu.sync_copy(m0_v, acc_sh.at[dst_v.at[k_self]], add=True)

        def ring(i, c2):
            for q in range(NBUF):
                slot(rows_b[q], av_b[q], bv_b[q], sems[q],
                     NBUF * i + q, NBUF * i + q + NBUF)
            return c2

        lax.fori_loop(0, IDXC // NBUF, ring, 0)
        return carry

    lax.fori_loop(0, TPB // IDXC, chunk, 0)
    plsc.subcore_barrier()

    ob = c * ACC_R + s * RPT

    def wb(j, carry):
        pltpu.sync_copy(acc_sh.at[pl.ds(base + j * BLK, BLK)],
                        out_hbm.at[pl.ds(ob + j * BLK, BLK)])
        return carry

    lax.fori_loop(0, RPT // BLK, wb, 0)
    pltpu.sync_copy(acc_sh.at[pl.ds(base + (RPT // BLK) * BLK, RPT % BLK)],
                    out_hbm.at[pl.ds(ob + (RPT // BLK) * BLK, RPT % BLK)])


_sc_edge = pl.kernel(
    _sc_edge_body,
    out_type=jax.ShapeDtypeStruct((2 * ACC_R, AW), jnp.float32),
    mesh=plsc.VectorSubcoreMesh(
        core_axis_name="c", subcore_axis_name="s",
        num_cores=2, num_subcores=16),
    compiler_params=pltpu.CompilerParams(
        use_tc_tiling_on_sc=False, needs_layout_passes=False),
    scratch_types=[
        pltpu.VMEM_SHARED((ACC_R, AW), jnp.float32),
        pltpu.VMEM((IDXC, BLK), jnp.int32),
        pltpu.VMEM((IDXC, BLK), jnp.int32),
        pltpu.VMEM((BLK, AW), jnp.float32),
        pltpu.VMEM((16,), jnp.float32),
        pltpu.VMEM((BLK, TW), jnp.bfloat16),
        pltpu.VMEM((BLK, TW), jnp.bfloat16),
        pltpu.VMEM((BLK, C), jnp.float32),
        pltpu.VMEM((BLK, C), jnp.float32),
        pltpu.VMEM((BLK, C), jnp.float32),
        pltpu.VMEM((BLK, C), jnp.float32),
        pltpu.SemaphoreType.DMA,
        pltpu.SemaphoreType.DMA,
    ],
)


def _att_mat(att):
    af = att.reshape(F)
    idx = jnp.arange(F) // C
    onehot = (idx[:, None] == jnp.arange(C)[None, :]).astype(jnp.float32)
    return af[:, None] * onehot


def _gat_layer(xw, a, b, m, srcp, dstp):
    g = jnp.maximum(m[0, 0] + m[0, 1], 0.0)
    gvec = jnp.full((16,), g, jnp.float32)
    t2b = (xw.reshape(N, H // 2, 2, C).transpose(0, 1, 3, 2)
           .reshape(N, F).astype(jnp.bfloat16))
    acc = _sc_edge(t2b, a, b, srcp, dstp, gvec)
    return acc[:N], acc[ACC_R:ACC_R + N]


def kernel(x, edge_index, W0, att_src0, att_dst0, b0,
           W1, att_src1, att_dst1, b1, Wc, bc):
    f32 = jnp.float32
    ss0 = _att_mat(att_src0)
    sd0 = _att_mat(att_dst0)
    ss1 = _att_mat(att_src1)
    sd1 = _att_mat(att_dst1)
    e8 = (jnp.arange(F)[None, :] // C == jnp.arange(H)[:, None]).astype(f32)

    src = edge_index[0]
    dst = edge_index[1]
    pad = EP - E
    srcp = jnp.concatenate(
        [src, jnp.zeros((pad,), jnp.int32)]).reshape(ROWS2D, BLK)
    dstp = jnp.concatenate(
        [dst, jnp.full((pad,), DUMMY, jnp.int32)]).reshape(ROWS2D, BLK)

    xw0, a0, bt0, m0 = _tc_proj(x, W0, ss0, sd0)
    p00, p01 = _gat_layer(xw0, a0, bt0, m0, srcp, dstp)

    xw1, a1, bt1, m1 = _tc_combine(
        p00, p01, e8, b0.reshape(1, F), W1, ss1, sd1, apply_elu=True)
    p10, p11 = _gat_layer(xw1, a1, bt1, m1, srcp, dstp)

    wc_pad = jnp.zeros((F, F), f32).at[:, :NCLS].set(Wc)
    bc_pad = jnp.zeros((1, F), f32).at[0, :NCLS].set(bc)
    logits = _tc_final(p10, p11, e8, b1.reshape(1, F), wc_pad, bc_pad)
    return logits[:, :NCLS]


# speedup vs baseline: 1.0438x; 1.0245x over previous
import jax
import jax.numpy as jnp
from jax import lax
from jax.experimental import pallas as pl
from jax.experimental.pallas import tpu as pltpu
from jax.experimental.pallas import tpu_sc as plsc

N = 10000
E = 320000
F = 128
H = 8
C = 16
NCLS = 40

NW = 32
BLK = 100
TPB = 100
IDXC = 10
NBUF = 2
EP = NW * TPB * BLK
ROWS2D = EP // BLK
ACC_R = 10112
DUMMY = N
RPT = ACC_R // 16
TW = F
AW = F + C

_BN = 2000
_GRID = N // _BN


def _finish_att(a, b, a_ref, b_ref, t2b, t2b_ref, m_ref, gv_ref):
    a_ref[...] = a
    b_ref[...] = b
    t2b_ref[...] = t2b
    cur = jnp.concatenate(
        [jnp.max(a).reshape(1, 1), jnp.max(b).reshape(1, 1)], axis=1)
    i = pl.program_id(0)

    @pl.when(i == 0)
    def _():
        m_ref[...] = cur

    @pl.when(i > 0)
    def _():
        m_ref[...] = jnp.maximum(m_ref[...], cur)

    @pl.when(i == _GRID - 1)
    def _():
        gv = jnp.maximum(m_ref[0, 0] + m_ref[0, 1], 0.0)
        gv_ref[...] = jnp.full((1, C), gv, jnp.float32)


def _tc_proj_body(x_ref, w_ref, ss_ref, sd_ref, p_ref,
                  a_ref, b_ref, t2b_ref, m_ref, gv_ref):
    xw = jnp.dot(x_ref[...], w_ref[...], preferred_element_type=jnp.float32)
    a = jnp.dot(xw, ss_ref[...], preferred_element_type=jnp.float32)
    b = jnp.dot(xw, sd_ref[...], preferred_element_type=jnp.float32)
    t2b = jnp.dot(
        xw, p_ref[...], preferred_element_type=jnp.float32).astype(
            jnp.bfloat16)
    _finish_att(a, b, a_ref, b_ref, t2b, t2b_ref, m_ref, gv_ref)


_ATT_OUT_SPECS = [
    pl.BlockSpec((_BN, C), lambda i: (i, 0)),
    pl.BlockSpec((_BN, C), lambda i: (i, 0)),
    pl.BlockSpec((_BN, F), lambda i: (i, 0)),
    pl.BlockSpec((1, 2), lambda i: (0, 0)),
    pl.BlockSpec((1, C), lambda i: (0, 0)),
]
_ATT_OUT_SHAPE = [
    jax.ShapeDtypeStruct((N, C), jnp.float32),
    jax.ShapeDtypeStruct((N, C), jnp.float32),
    jax.ShapeDtypeStruct((N, F), jnp.bfloat16),
    jax.ShapeDtypeStruct((1, 2), jnp.float32),
    jax.ShapeDtypeStruct((1, C), jnp.float32),
]


def _tc_proj(x, w, ss, sd, p):
    fw = x.shape[1]
    return pl.pallas_call(
        _tc_proj_body,
        grid=(_GRID,),
        in_specs=[
            pl.BlockSpec((_BN, fw), lambda i: (i, 0)),
            pl.BlockSpec((fw, F), lambda i: (0, 0)),
            pl.BlockSpec((F, C), lambda i: (0, 0)),
            pl.BlockSpec((F, C), lambda i: (0, 0)),
            pl.BlockSpec((F, F), lambda i: (0, 0)),
        ],
        out_specs=_ATT_OUT_SPECS,
        out_shape=_ATT_OUT_SHAPE,
    )(x, w, ss, sd, p)


def _tc_comb_body(p0_ref, p1_ref, e8_ref, bias_ref, w_ref,
                  ss_ref, sd_ref, p_ref,
                  a_ref, b_ref, t2b_ref, m_ref, gv_ref):
    acc = p0_ref[...] + p1_ref[...]
    num = acc[:, :F]
    den = acc[:, F:F + H]
    dene = jnp.dot(den, e8_ref[...], preferred_element_type=jnp.float32)
    h = num / (dene + 1e-16) + bias_ref[...]
    h = jnp.where(h > 0.0, h, jnp.exp(h) - 1.0)
    xw = jnp.dot(h, w_ref[...], preferred_element_type=jnp.float32)
    a = jnp.dot(xw, ss_ref[...], preferred_element_type=jnp.float32)
    b = jnp.dot(xw, sd_ref[...], preferred_element_type=jnp.float32)
    t2b = jnp.dot(
        xw, p_ref[...], preferred_element_type=jnp.float32).astype(
            jnp.bfloat16)
    _finish_att(a, b, a_ref, b_ref, t2b, t2b_ref, m_ref, gv_ref)


def _tc_combine(p0, p1, e8, bias, w, ss, sd, p):
    return pl.pallas_call(
        _tc_comb_body,
        grid=(_GRID,),
        in_specs=[
            pl.BlockSpec((_BN, AW), lambda i: (i, 0)),
            pl.BlockSpec((_BN, AW), lambda i: (i, 0)),
            pl.BlockSpec((H, F), lambda i: (0, 0)),
            pl.BlockSpec((1, F), lambda i: (0, 0)),
            pl.BlockSpec((F, F), lambda i: (0, 0)),
            pl.BlockSpec((F, C), lambda i: (0, 0)),
            pl.BlockSpec((F, C), lambda i: (0, 0)),
            pl.BlockSpec((F, F), lambda i: (0, 0)),
        ],
        out_specs=_ATT_OUT_SPECS,
        out_shape=_ATT_OUT_SHAPE,
    )(p0, p1, e8, bias, w, ss, sd, p)


def _tc_final_body(p0_ref, p1_ref, e8_ref, bias_ref, w_ref, bc_ref, out_ref):
    acc = p0_ref[...] + p1_ref[...]
    num = acc[:, :F]
    den = acc[:, F:F + H]
    dene = jnp.dot(den, e8_ref[...], preferred_element_type=jnp.float32)
    h = num / (dene + 1e-16) + bias_ref[...]
    out_ref[...] = (
        jnp.dot(h, w_ref[...], preferred_element_type=jnp.float32)
        + bc_ref[...])


def _tc_final(p0, p1, e8, bias, wc_pad, bc_pad):
    return pl.pallas_call(
        _tc_final_body,
        grid=(_GRID,),
        in_specs=[
            pl.BlockSpec((_BN, AW), lambda i: (i, 0)),
            pl.BlockSpec((_BN, AW), lambda i: (i, 0)),
            pl.BlockSpec((H, F), lambda i: (0, 0)),
            pl.BlockSpec((1, F), lambda i: (0, 0)),
            pl.BlockSpec((F, F), lambda i: (0, 0)),
            pl.BlockSpec((1, F), lambda i: (0, 0)),
        ],
        out_specs=pl.BlockSpec((_BN, F), lambda i: (i, 0)),
        out_shape=jax.ShapeDtypeStruct((N, F), jnp.float32),
    )(p0, p1, e8, bias, wc_pad, bc_pad)


def _sc_edge_body(t2_hbm, at_hbm, bt_hbm, src_hbm, dst_hbm, g_hbm, out_hbm,
                  acc_sh, src_v, dst_v, m0_v, g_v,
                  rows0_v, rows1_v, av0_v, av1_v, bv0_v, bv1_v,
                  sem0, sem1):
    c = lax.axis_index("c")
    s = lax.axis_index("s")
    wid = s * 2 + c
    rows_b = [rows0_v, rows1_v]
    av_b = [av0_v, av1_v]
    bv_b = [bv0_v, bv1_v]
    sems = [sem0, sem1]

    zero16 = jnp.zeros((16,), jnp.float32)

    def zrow(i, carry):
        for g in range(AW // 16):
            m0_v[i, pl.ds(g * 16, 16)] = zero16
        return carry

    lax.fori_loop(0, BLK, zrow, 0)
    base = s * RPT

    def zacc(j, carry):
        pltpu.sync_copy(m0_v, acc_sh.at[pl.ds(base + j * BLK, BLK)])
        return carry

    lax.fori_loop(0, RPT // BLK, zacc, 0)
    pltpu.sync_copy(m0_v.at[pl.ds(0, RPT % BLK)],
                    acc_sh.at[pl.ds(base + (RPT // BLK) * BLK, RPT % BLK)])
    plsc.subcore_barrier()

    pltpu.sync_copy(g_hbm, g_v)
    gv = g_v[...]
    maskv = jnp.where(lax.iota(jnp.int32, 16) < H, 1.0, 0.0)

    def _compute(rows_v, av_v, bv_v, m_v):
        def edge(b, inner):
            alpha = av_v[b, :] + bv_v[b, :]
            alpha = jnp.where(alpha > 0.0, alpha, alpha * 0.2)
            w = jnp.exp(alpha - gv)
            for g in range(H // 2):
                v = rows_v[b, pl.ds(32 * g, 32)]
                vi = plsc.bitcast(v, jnp.int32)
                a16 = plsc.bitcast(lax.shift_left(vi, 16), jnp.float32)
                b16 = plsc.bitcast(vi & jnp.int32(-65536), jnp.float32)
                m_v[b, pl.ds(2 * g * 16, 16)] = a16 * w[2 * g]
                m_v[b, pl.ds((2 * g + 1) * 16, 16)] = b16 * w[2 * g + 1]
            m_v[b, pl.ds(F, 16)] = w * maskv
            return inner

        lax.fori_loop(0, BLK, edge, 0, unroll=8)

    def chunk(kk, carry):
        row0 = wid * TPB + kk * IDXC
        pltpu.sync_copy(src_hbm.at[pl.ds(row0, IDXC)], src_v)
        pltpu.sync_copy(dst_hbm.at[pl.ds(row0, IDXC)], dst_v)
        for q in range(NBUF):
            pltpu.async_copy(t2_hbm.at[src_v.at[q]], rows_b[q], sems[q])
            pltpu.async_copy(at_hbm.at[src_v.at[q]], av_b[q], sems[q])
            pltpu.async_copy(bt_hbm.at[dst_v.at[q]], bv_b[q], sems[q])

        def slot(rows_v, av_v, bv_v, sem, k_self, k_next):
            pltpu.make_async_copy(
                t2_hbm.at[src_v.at[k_self]], rows_v, sem).wait()
            pltpu.make_async_copy(
                at_hbm.at[src_v.at[k_self]], av_v, sem).wait()
            pltpu.make_async_copy(
                bt_hbm.at[dst_v.at[k_self]], bv_v, sem).wait()

            _compute(rows_v, av_v, bv_v, m0_v)

            @pl.when(k_next < IDXC)
            def _():
                pltpu.async_copy(t2_hbm.at[src_v.at[k_next]], rows_v, sem)
                pltpu.async_copy(at_hbm.at[src_v.at[k_next]], av_v, sem)
                pltpu.async_copy(bt_hbm.at[dst_v.at[k_next]], bv_v, sem)

            pltpu.sync_copy(m0_v, acc_sh.at[dst_v.at[k_self]], add=True)

        def ring(i, c2):
            for q in range(NBUF):
                slot(rows_b[q], av_b[q], bv_b[q], sems[q],
                     NBUF * i + q, NBUF * i + q + NBUF)
            return c2

        lax.fori_loop(0, IDXC // NBUF, ring, 0)
        return carry

    lax.fori_loop(0, TPB // IDXC, chunk, 0)
    plsc.subcore_barrier()

    ob = c * ACC_R + s * RPT

    def wb(j, carry):
        pltpu.sync_copy(acc_sh.at[pl.ds(base + j * BLK, BLK)],
                        out_hbm.at[pl.ds(ob + j * BLK, BLK)])
        return carry

    lax.fori_loop(0, RPT // BLK, wb, 0)
    pltpu.sync_copy(acc_sh.at[pl.ds(base + (RPT // BLK) * BLK, RPT % BLK)],
                    out_hbm.at[pl.ds(ob + (RPT // BLK) * BLK, RPT % BLK)])


_sc_edge = pl.kernel(
    _sc_edge_body,
    out_type=jax.ShapeDtypeStruct((2 * ACC_R, AW), jnp.float32),
    mesh=plsc.VectorSubcoreMesh(
        core_axis_name="c", subcore_axis_name="s",
        num_cores=2, num_subcores=16),
    compiler_params=pltpu.CompilerParams(
        use_tc_tiling_on_sc=False, needs_layout_passes=False),
    scratch_types=[
        pltpu.VMEM_SHARED((ACC_R, AW), jnp.float32),
        pltpu.VMEM((IDXC, BLK), jnp.int32),
        pltpu.VMEM((IDXC, BLK), jnp.int32),
        pltpu.VMEM((BLK, AW), jnp.float32),
        pltpu.VMEM((16,), jnp.float32),
        pltpu.VMEM((BLK, TW), jnp.bfloat16),
        pltpu.VMEM((BLK, TW), jnp.bfloat16),
        pltpu.VMEM((BLK, C), jnp.float32),
        pltpu.VMEM((BLK, C), jnp.float32),
        pltpu.VMEM((BLK, C), jnp.float32),
        pltpu.VMEM((BLK, C), jnp.float32),
        pltpu.SemaphoreType.DMA,
        pltpu.SemaphoreType.DMA,
    ],
)


def _att_mat(att):
    af = att.reshape(F)
    idx = jnp.arange(F) // C
    onehot = (idx[:, None] == jnp.arange(C)[None, :]).astype(jnp.float32)
    return af[:, None] * onehot


def _gat_layer(a, b, t2b, gv, srcp, dstp):
    acc = _sc_edge(t2b, a, b, srcp, dstp, gv.reshape(C))
    return acc[:N], acc[ACC_R:ACC_R + N]


def kernel(x, edge_index, W0, att_src0, att_dst0, b0,
           W1, att_src1, att_dst1, b1, Wc, bc):
    f32 = jnp.float32
    ss0 = _att_mat(att_src0)
    sd0 = _att_mat(att_dst0)
    ss1 = _att_mat(att_src1)
    sd1 = _att_mat(att_dst1)
    e8 = (jnp.arange(F)[None, :] // C == jnp.arange(H)[:, None]).astype(f32)
    cols = jnp.arange(F)
    orig = (2 * (cols // 32) + cols % 2) * C + (cols % 32) // 2
    perm = (jnp.arange(F)[:, None] == orig[None, :]).astype(f32)

    src = edge_index[0]
    dst = edge_index[1]
    pad = EP - E
    srcp = jnp.concatenate(
        [src, jnp.zeros((pad,), jnp.int32)]).reshape(ROWS2D, BLK)
    dstp = jnp.concatenate(
        [dst, jnp.full((pad,), DUMMY, jnp.int32)]).reshape(ROWS2D, BLK)

    a0, bt0, t2b0, _, gv0 = _tc_proj(x, W0, ss0, sd0, perm)
    p00, p01 = _gat_layer(a0, bt0, t2b0, gv0, srcp, dstp)

    a1, bt1, t2b1, _, gv1 = _tc_combine(
        p00, p01, e8, b0.reshape(1, F), W1, ss1, sd1, perm)
    p10, p11 = _gat_layer(a1, bt1, t2b1, gv1, srcp, dstp)

    wc_pad = jnp.zeros((F, F), f32).at[:, :NCLS].set(Wc)
    bc_pad = jnp.zeros((1, F), f32).at[0, :NCLS].set(bc)
    logits = _tc_final(p10, p11, e8, b1.reshape(1, F), wc_pad, bc_pad)
    return logits[:, :NCLS]


# speedup vs baseline: 1.0453x; 1.0015x over previous
import jax
import jax.numpy as jnp
from jax import lax
from jax.experimental import pallas as pl
from jax.experimental.pallas import tpu as pltpu
from jax.experimental.pallas import tpu_sc as plsc

N = 10000
E = 320000
F = 128
H = 8
C = 16
NCLS = 40

NW = 32
BLK = 100
TPB = 100
IDXC = 10
NBUF = 2
EP = NW * TPB * BLK
ROWS2D = EP // BLK
ACC_R = 10112
DUMMY = N
RPT = ACC_R // 16
TW = F
AW = F + C

_BN = 2000
_GRID = N // _BN


def _finish_att(a, b, a_ref, b_ref, t2b, t2b_ref, m_ref, gv_ref):
    a_ref[...] = a
    b_ref[...] = b
    t2b_ref[...] = t2b
    cur = jnp.concatenate(
        [jnp.max(a).reshape(1, 1), jnp.max(b).reshape(1, 1)], axis=1)
    i = pl.program_id(0)

    @pl.when(i == 0)
    def _():
        m_ref[...] = cur

    @pl.when(i > 0)
    def _():
        m_ref[...] = jnp.maximum(m_ref[...], cur)

    @pl.when(i == _GRID - 1)
    def _():
        gv = jnp.maximum(m_ref[0, 0] + m_ref[0, 1], 0.0)
        gv_ref[...] = jnp.full((1, C), gv, jnp.float32)


def _tc_proj_body(x_ref, w_ref, ss_ref, sd_ref, p_ref,
                  a_ref, b_ref, t2b_ref, m_ref, gv_ref):
    xw = jnp.dot(x_ref[...], w_ref[...], preferred_element_type=jnp.float32)
    a = jnp.dot(xw, ss_ref[...], preferred_element_type=jnp.float32)
    b = jnp.dot(xw, sd_ref[...], preferred_element_type=jnp.float32)
    t2b = jnp.dot(
        xw, p_ref[...], preferred_element_type=jnp.float32).astype(
            jnp.bfloat16)
    _finish_att(a, b, a_ref, b_ref, t2b, t2b_ref, m_ref, gv_ref)


_ATT_OUT_SPECS = [
    pl.BlockSpec((_BN, C), lambda i: (i, 0)),
    pl.BlockSpec((_BN, C), lambda i: (i, 0)),
    pl.BlockSpec((_BN, F), lambda i: (i, 0)),
    pl.BlockSpec((1, 2), lambda i: (0, 0)),
    pl.BlockSpec((1, C), lambda i: (0, 0)),
]
_ATT_OUT_SHAPE = [
    jax.ShapeDtypeStruct((N, C), jnp.float32),
    jax.ShapeDtypeStruct((N, C), jnp.float32),
    jax.ShapeDtypeStruct((N, F), jnp.bfloat16),
    jax.ShapeDtypeStruct((1, 2), jnp.float32),
    jax.ShapeDtypeStruct((1, C), jnp.float32),
]


def _tc_proj(x, w, ss, sd, p):
    fw = x.shape[1]
    return pl.pallas_call(
        _tc_proj_body,
        grid=(_GRID,),
        in_specs=[
            pl.BlockSpec((_BN, fw), lambda i: (i, 0)),
            pl.BlockSpec((fw, F), lambda i: (0, 0)),
            pl.BlockSpec((F, C), lambda i: (0, 0)),
            pl.BlockSpec((F, C), lambda i: (0, 0)),
            pl.BlockSpec((F, F), lambda i: (0, 0)),
        ],
        out_specs=_ATT_OUT_SPECS,
        out_shape=_ATT_OUT_SHAPE,
    )(x, w, ss, sd, p)


def _tc_comb_body(p0_ref, p1_ref, e8_ref, bias_ref, w_ref,
                  ss_ref, sd_ref, p_ref,
                  a_ref, b_ref, t2b_ref, m_ref, gv_ref):
    acc = p0_ref[...] + p1_ref[...]
    num = acc[:, :F]
    den = acc[:, F:F + H]
    dene = jnp.dot(den, e8_ref[...], preferred_element_type=jnp.float32)
    h = num / (dene + 1e-16) + bias_ref[...]
    h = jnp.where(h > 0.0, h, jnp.exp(h) - 1.0)
    xw = jnp.dot(h, w_ref[...], preferred_element_type=jnp.float32)
    a = jnp.dot(xw, ss_ref[...], preferred_element_type=jnp.float32)
    b = jnp.dot(xw, sd_ref[...], preferred_element_type=jnp.float32)
    t2b = jnp.dot(
        xw, p_ref[...], preferred_element_type=jnp.float32).astype(
            jnp.bfloat16)
    _finish_att(a, b, a_ref, b_ref, t2b, t2b_ref, m_ref, gv_ref)


def _tc_combine(p0, p1, e8, bias, w, ss, sd, p):
    return pl.pallas_call(
        _tc_comb_body,
        grid=(_GRID,),
        in_specs=[
            pl.BlockSpec((_BN, AW), lambda i: (i, 0)),
            pl.BlockSpec((_BN, AW), lambda i: (i, 0)),
            pl.BlockSpec((H, F), lambda i: (0, 0)),
            pl.BlockSpec((1, F), lambda i: (0, 0)),
            pl.BlockSpec((F, F), lambda i: (0, 0)),
            pl.BlockSpec((F, C), lambda i: (0, 0)),
            pl.BlockSpec((F, C), lambda i: (0, 0)),
            pl.BlockSpec((F, F), lambda i: (0, 0)),
        ],
        out_specs=_ATT_OUT_SPECS,
        out_shape=_ATT_OUT_SHAPE,
    )(p0, p1, e8, bias, w, ss, sd, p)


def _tc_final_body(p0_ref, p1_ref, e8_ref, bias_ref, w_ref, bc_ref, out_ref):
    acc = p0_ref[...] + p1_ref[...]
    num = acc[:, :F]
    den = acc[:, F:F + H]
    dene = jnp.dot(den, e8_ref[...], preferred_element_type=jnp.float32)
    h = num / (dene + 1e-16) + bias_ref[...]
    out_ref[...] = (
        jnp.dot(h, w_ref[...], preferred_element_type=jnp.float32)
        + bc_ref[...])


def _tc_final(p0, p1, e8, bias, wc_pad, bc_pad):
    return pl.pallas_call(
        _tc_final_body,
        grid=(_GRID,),
        in_specs=[
            pl.BlockSpec((_BN, AW), lambda i: (i, 0)),
            pl.BlockSpec((_BN, AW), lambda i: (i, 0)),
            pl.BlockSpec((H, F), lambda i: (0, 0)),
            pl.BlockSpec((1, F), lambda i: (0, 0)),
            pl.BlockSpec((F, F), lambda i: (0, 0)),
            pl.BlockSpec((1, F), lambda i: (0, 0)),
        ],
        out_specs=pl.BlockSpec((_BN, F), lambda i: (i, 0)),
        out_shape=jax.ShapeDtypeStruct((N, F), jnp.float32),
    )(p0, p1, e8, bias, wc_pad, bc_pad)


def _sc_edge_body(t2_hbm, at_hbm, bt_hbm, src_hbm, dst_hbm, g_hbm, out_hbm,
                  acc_sh, src_v, dst_v, m0_v, g_v,
                  rows0_v, rows1_v, av0_v, av1_v, bv0_v, bv1_v,
                  sem0, sem1):
    c = lax.axis_index("c")
    s = lax.axis_index("s")
    wid = s * 2 + c
    rows_b = [rows0_v, rows1_v]
    av_b = [av0_v, av1_v]
    bv_b = [bv0_v, bv1_v]
    sems = [sem0, sem1]

    zero16 = jnp.zeros((16,), jnp.float32)

    def zrow(i, carry):
        for g in range(AW // 16):
            m0_v[i, pl.ds(g * 16, 16)] = zero16
        return carry

    lax.fori_loop(0, BLK, zrow, 0)
    base = s * RPT

    def zacc(j, carry):
        pltpu.async_copy(m0_v, acc_sh.at[pl.ds(base + j * BLK, BLK)], sem0)
        return carry

    lax.fori_loop(0, RPT // BLK, zacc, 0)
    pltpu.async_copy(m0_v.at[pl.ds(0, RPT % BLK)],
                     acc_sh.at[pl.ds(base + (RPT // BLK) * BLK, RPT % BLK)],
                     sem0)

    def zacc_w(j, carry):
        pltpu.make_async_copy(
            m0_v, acc_sh.at[pl.ds(base + j * BLK, BLK)], sem0).wait()
        return carry

    lax.fori_loop(0, RPT // BLK, zacc_w, 0)
    pltpu.make_async_copy(
        m0_v.at[pl.ds(0, RPT % BLK)],
        acc_sh.at[pl.ds(base + (RPT // BLK) * BLK, RPT % BLK)], sem0).wait()
    plsc.subcore_barrier()

    pltpu.sync_copy(g_hbm, g_v)
    gv = g_v[...]
    maskv = jnp.where(lax.iota(jnp.int32, 16) < H, 1.0, 0.0)

    def _compute(rows_v, av_v, bv_v, m_v):
        def edge(b, inner):
            alpha = av_v[b, :] + bv_v[b, :]
            alpha = jnp.where(alpha > 0.0, alpha, alpha * 0.2)
            w = jnp.exp(alpha - gv)
            for g in range(H // 2):
                v = rows_v[b, pl.ds(32 * g, 32)]
                vi = plsc.bitcast(v, jnp.int32)
                a16 = plsc.bitcast(lax.shift_left(vi, 16), jnp.float32)
                b16 = plsc.bitcast(vi & jnp.int32(-65536), jnp.float32)
                m_v[b, pl.ds(2 * g * 16, 16)] = a16 * w[2 * g]
                m_v[b, pl.ds((2 * g + 1) * 16, 16)] = b16 * w[2 * g + 1]
            m_v[b, pl.ds(F, 16)] = w * maskv
            return inner

        lax.fori_loop(0, BLK, edge, 0, unroll=8)

    def chunk(kk, carry):
        row0 = wid * TPB + kk * IDXC
        pltpu.sync_copy(src_hbm.at[pl.ds(row0, IDXC)], src_v)
        pltpu.sync_copy(dst_hbm.at[pl.ds(row0, IDXC)], dst_v)
        for q in range(NBUF):
            pltpu.async_copy(t2_hbm.at[src_v.at[q]], rows_b[q], sems[q])
            pltpu.async_copy(at_hbm.at[src_v.at[q]], av_b[q], sems[q])
            pltpu.async_copy(bt_hbm.at[dst_v.at[q]], bv_b[q], sems[q])

        def slot(rows_v, av_v, bv_v, sem, k_self, k_next):
            pltpu.make_async_copy(
                t2_hbm.at[src_v.at[k_self]], rows_v, sem).wait()
            pltpu.make_async_copy(
                at_hbm.at[src_v.at[k_self]], av_v, sem).wait()
            pltpu.make_async_copy(
                bt_hbm.at[dst_v.at[k_self]], bv_v, sem).wait()

            _compute(rows_v, av_v, bv_v, m0_v)

            @pl.when(k_next < IDXC)
            def _():
                pltpu.async_copy(t2_hbm.at[src_v.at[k_next]], rows_v, sem)
                pltpu.async_copy(at_hbm.at[src_v.at[k_next]], av_v, sem)
                pltpu.async_copy(bt_hbm.at[dst_v.at[k_next]], bv_v, sem)

            pltpu.sync_copy(m0_v, acc_sh.at[dst_v.at[k_self]], add=True)

        def ring(i, c2):
            for q in range(NBUF):
                slot(rows_b[q], av_b[q], bv_b[q], sems[q],
                     NBUF * i + q, NBUF * i + q + NBUF)
            return c2

        lax.fori_loop(0, IDXC // NBUF, ring, 0)
        return carry

    lax.fori_loop(0, TPB // IDXC, chunk, 0)
    plsc.subcore_barrier()

    ob = c * ACC_R + s * RPT

    def wb(j, carry):
        pltpu.async_copy(acc_sh.at[pl.ds(base + j * BLK, BLK)],
                         out_hbm.at[pl.ds(ob + j * BLK, BLK)], sem0)
        return carry

    lax.fori_loop(0, RPT // BLK, wb, 0)
    pltpu.async_copy(acc_sh.at[pl.ds(base + (RPT // BLK) * BLK, RPT % BLK)],
                     out_hbm.at[pl.ds(ob + (RPT // BLK) * BLK, RPT % BLK)],
                     sem0)

    def wb_w(j, carry):
        pltpu.make_async_copy(
            acc_sh.at[pl.ds(base + j * BLK, BLK)],
            out_hbm.at[pl.ds(ob + j * BLK, BLK)], sem0).wait()
        return carry

    lax.fori_loop(0, RPT // BLK, wb_w, 0)
    pltpu.make_async_copy(
        acc_sh.at[pl.ds(base + (RPT // BLK) * BLK, RPT % BLK)],
        out_hbm.at[pl.ds(ob + (RPT // BLK) * BLK, RPT % BLK)], sem0).wait()


_sc_edge = pl.kernel(
    _sc_edge_body,
    out_type=jax.ShapeDtypeStruct((2 * ACC_R, AW), jnp.float32),
    mesh=plsc.VectorSubcoreMesh(
        core_axis_name="c", subcore_axis_name="s",
        num_cores=2, num_subcores=16),
    compiler_params=pltpu.CompilerParams(
        use_tc_tiling_on_sc=False, needs_layout_passes=False),
    scratch_types=[
        pltpu.VMEM_SHARED((ACC_R, AW), jnp.float32),
        pltpu.VMEM((IDXC, BLK), jnp.int32),
        pltpu.VMEM((IDXC, BLK), jnp.int32),
        pltpu.VMEM((BLK, AW), jnp.float32),
        pltpu.VMEM((16,), jnp.float32),
        pltpu.VMEM((BLK, TW), jnp.bfloat16),
        pltpu.VMEM((BLK, TW), jnp.bfloat16),
        pltpu.VMEM((BLK, C), jnp.float32),
        pltpu.VMEM((BLK, C), jnp.float32),
        pltpu.VMEM((BLK, C), jnp.float32),
        pltpu.VMEM((BLK, C), jnp.float32),
        pltpu.SemaphoreType.DMA,
        pltpu.SemaphoreType.DMA,
    ],
)


def _att_mat(att):
    af = att.reshape(F)
    idx = jnp.arange(F) // C
    onehot = (idx[:, None] == jnp.arange(C)[None, :]).astype(jnp.float32)
    return af[:, None] * onehot


def _gat_layer(a, b, t2b, gv, srcp, dstp):
    acc = _sc_edge(t2b, a, b, srcp, dstp, gv.reshape(C))
    return acc[:N], acc[ACC_R:ACC_R + N]


def kernel(x, edge_index, W0, att_src0, att_dst0, b0,
           W1, att_src1, att_dst1, b1, Wc, bc):
    f32 = jnp.float32
    ss0 = _att_mat(att_src0)
    sd0 = _att_mat(att_dst0)
    ss1 = _att_mat(att_src1)
    sd1 = _att_mat(att_dst1)
    e8 = (jnp.arange(F)[None, :] // C == jnp.arange(H)[:, None]).astype(f32)
    cols = jnp.arange(F)
    orig = (2 * (cols // 32) + cols % 2) * C + (cols % 32) // 2
    perm = (jnp.arange(F)[:, None] == orig[None, :]).astype(f32)

    src = edge_index[0]
    dst = edge_index[1]
    pad = EP - E
    srcp = jnp.concatenate(
        [src, jnp.zeros((pad,), jnp.int32)]).reshape(ROWS2D, BLK)
    dstp = jnp.concatenate(
        [dst, jnp.full((pad,), DUMMY, jnp.int32)]).reshape(ROWS2D, BLK)

    a0, bt0, t2b0, _, gv0 = _tc_proj(x, W0, ss0, sd0, perm)
    p00, p01 = _gat_layer(a0, bt0, t2b0, gv0, srcp, dstp)

    a1, bt1, t2b1, _, gv1 = _tc_combine(
        p00, p01, e8, b0.reshape(1, F), W1, ss1, sd1, perm)
    p10, p11 = _gat_layer(a1, bt1, t2b1, gv1, srcp, dstp)

    wc_pad = jnp.zeros((F, F), f32).at[:, :NCLS].set(Wc)
    bc_pad = jnp.zeros((1, F), f32).at[0, :NCLS].set(bc)
    logits = _tc_final(p10, p11, e8, b1.reshape(1, F), wc_pad, bc_pad)
    return logits[:, :NCLS]
